# baseline (speedup 1.0000x reference)
import functools

import jax
import jax.numpy as jnp
from jax import lax
from jax.experimental import pallas as pl
from jax.experimental.pallas import tpu as pltpu, tpu_sc as plsc

_BATCH = 16384
_F = 26
_D = 64
_C = 1000
_NW = 32
_BPW = _BATCH // _NW
_CH = 128
_NCH = _BPW // _CH
_NSTEP = _F * _NCH
_NSLOT = 4
_NCHG = _BATCH // _CH


def _qr_split(v):
    q = (v.astype(jnp.float32) * jnp.float32(1.0 / _C)).astype(jnp.int32)
    r = v - q * _C
    too_big = r >= _C
    too_small = r < 0
    q = jnp.where(too_big, q + 1, jnp.where(too_small, q - 1, q))
    r = jnp.where(too_big, r - _C, jnp.where(too_small, r + _C, r))
    return q, r


def _body(idx_hbm, wq_hbm, wr_hbm, out_hbm, qidx, ridx, gq, gr, prod, *sems):
    semq = sems[0:_NSLOT]
    semr = sems[_NSLOT:2 * _NSLOT]
    semo = sems[2 * _NSLOT:3 * _NSLOT]
    wid = lax.axis_index("s") * 2 + lax.axis_index("c")
    row0 = wid * _BPW

    pltpu.sync_copy(idx_hbm.at[:, pl.ds(wid * _NCH, _NCH), :], qidx)

    @pl.loop(0, _F)
    def _prep(f):
        off = jnp.full((16,), f * _C, jnp.int32)
        for ch in range(_NCH):
            for j in range(_CH // 16):
                v = qidx[f, ch, pl.ds(j * 16, 16)]
                q, r = _qr_split(v)
                qidx[f, ch, pl.ds(j * 16, 16)] = q + off
                ridx[f, ch, pl.ds(j * 16, 16)] = r + off

    def _fire(s, slot):
        f = s // _NCH
        ch = s - f * _NCH
        pltpu.async_copy(wq_hbm.at[qidx.at[f, ch]], gq.at[slot], semq[slot])
        pltpu.async_copy(wr_hbm.at[ridx.at[f, ch]], gr.at[slot], semr[slot])

    def _wait_gather(slot):
        pltpu.make_async_copy(wq_hbm.at[pl.ds(0, _CH)], gq.at[slot], semq[slot]).wait()
        pltpu.make_async_copy(wr_hbm.at[pl.ds(0, _CH)], gr.at[slot], semr[slot]).wait()

    def _wait_out(slot):
        pltpu.make_async_copy(
            prod.at[slot], out_hbm.at[pl.ds(0, _CH), pl.ds(0, _D)], semo[slot]
        ).wait()

    for b in range(_NSLOT - 1):
        _fire(b, b)

    @pl.loop(0, _NSTEP, step=_NSLOT)
    def _main(s0):
        f = s0 // _NCH
        for b in range(_NSLOT):
            s3 = s0 + b + (_NSLOT - 1)

            @pl.when(s3 < _NSTEP)
            def _():
                _fire(s3, (b + _NSLOT - 1) % _NSLOT)

            @pl.when(s0 > 0)
            def _():
                _wait_out(b)

            _wait_gather(b)

            gqb = gq.at[b]
            grb = gr.at[b]
            prb = prod.at[b]

            @pl.loop(0, _CH)
            def _mul(i):
                for c in range(_D // 16):
                    prb[i, pl.ds(c * 16, 16)] = (
                        gqb[i, pl.ds(c * 16, 16)] * grb[i, pl.ds(c * 16, 16)]
                    )

            pltpu.async_copy(
                prod.at[b],
                out_hbm.at[pl.ds(row0 + b * _CH, _CH), pl.ds(f * _D, _D)],
                semo[b],
            )

    for b in range(_NSLOT):
        _wait_out(b)


@jax.jit
def _qr_embedding(idx_r, wq_flat, wr_flat):
    mesh = plsc.VectorSubcoreMesh(core_axis_name="c", subcore_axis_name="s")
    return pl.kernel(
        _body,
        out_type=jax.ShapeDtypeStruct((_BATCH, _F * _D), jnp.float32),
        mesh=mesh,
        compiler_params=pltpu.CompilerParams(use_tc_tiling_on_sc=False),
        scratch_types=[
            pltpu.VMEM((_F, _NCH, _CH), jnp.int32),
            pltpu.VMEM((_F, _NCH, _CH), jnp.int32),
            pltpu.VMEM((_NSLOT, _CH, _D), jnp.float32),
            pltpu.VMEM((_NSLOT, _CH, _D), jnp.float32),
            pltpu.VMEM((_NSLOT, _CH, _D), jnp.float32),
        ] + [pltpu.SemaphoreType.DMA] * (3 * _NSLOT),
    )(idx_r, wq_flat, wr_flat)


def kernel(indices, W_q, W_r):
    idx_r = indices.T.reshape(_F, _NCHG, _CH)
    wq_flat = W_q.reshape(_F * _C, _D)
    wr_flat = W_r.reshape(_F * _C, _D)
    return _qr_embedding(idx_r, wq_flat, wr_flat)
